# baseline (speedup 1.0000x reference)
import functools

import jax
import jax.numpy as jnp
from jax import lax
from jax.experimental import pallas as pl
from jax.experimental.pallas import tpu as pltpu
from jax.experimental.pallas import tpu_sc as plsc

T = 2048
K_TOP = 2
E = 8
INTER = 4096
HID = 1024
R = T * K_TOP

BLK = 256
NB = R // BLK
G = NB + E
KBLK = 2048
NK = INTER // KBLK
PCH = 512
NPC = R // PCH

NW = 32
RPW = R // NW
TPW = T // NW
GCH = 16


def _plan_body(logits_ref, perm_ref, idx0_ref, idx1_ref, wts_ref, tbl_ref):
    logits = logits_ref[...]
    col = lax.broadcasted_iota(jnp.int32, (T, 128), 1)
    m = jnp.max(logits, axis=-1, keepdims=True)
    ex = jnp.exp(logits - m)
    probs = ex / jnp.sum(ex, axis=-1, keepdims=True)

    m1 = jnp.max(probs, axis=-1, keepdims=True)
    i1 = jnp.min(jnp.where(probs == m1, col, 128), axis=-1, keepdims=True)
    probs2 = jnp.where(col == i1, -1.0, probs)
    m2 = jnp.max(probs2, axis=-1, keepdims=True)
    i2 = jnp.min(jnp.where(probs2 == m2, col, 128), axis=-1, keepdims=True)

    oh1 = (col == i1).astype(jnp.float32)
    oh2 = (col == i2).astype(jnp.float32)
    oh = oh1 + oh2

    nt = T // 128
    ohr = oh.reshape(nt, 128, 128)
    r1 = lax.broadcasted_iota(jnp.int32, (128, 128), 0)
    c1 = lax.broadcasted_iota(jnp.int32, (128, 128), 1)
    ltri1 = jnp.broadcast_to((r1 >= c1).astype(jnp.float32), (nt, 128, 128))
    ci = lax.dot_general(ltri1, ohr, (((2,), (1,)), ((0,), (0,))),
                         preferred_element_type=jnp.float32)
    tile_sum = jnp.sum(ohr, axis=1)
    r2 = lax.broadcasted_iota(jnp.int32, (nt, nt), 0)
    c2 = lax.broadcasted_iota(jnp.int32, (nt, nt), 1)
    sltri2 = (r2 > c2).astype(jnp.float32)
    tile_pre = lax.dot(sltri2, tile_sum,
                       preferred_element_type=jnp.float32)
    c_incl = (ci + tile_pre[:, None, :]).reshape(T, 128)
    c_excl = c_incl - oh

    counts = jnp.sum(oh, axis=0, keepdims=True)

    lr = lax.broadcasted_iota(jnp.int32, (128, 128), 0)
    lc = lax.broadcasted_iota(jnp.int32, (128, 128), 1)
    sl = (lr < lc).astype(jnp.float32)
    offs_e = lax.dot(counts, sl, precision=lax.Precision.HIGHEST,
                     preferred_element_type=jnp.float32)

    lane = lax.broadcasted_iota(jnp.int32, (1, 128), 1)
    lanef = lane.astype(jnp.float32)
    blk_f = jnp.float32(BLK)
    lo_f = offs_e
    hi_f = offs_e + counts
    blo = jnp.floor(lo_f / blk_f)
    bhi = jnp.floor((hi_f - 1.0) / blk_f)
    slots = jnp.where(counts > 0.0, bhi - blo + 1.0, 0.0)
    sstart = lax.dot(slots, sl, precision=lax.Precision.HIGHEST,
                     preferred_element_type=jnp.float32)
    total = jnp.sum(slots)
    eid = jnp.zeros((1, 128), jnp.float32)
    bid = jnp.zeros((1, 128), jnp.float32)
    for e in range(E):
        se = jnp.sum(jnp.where(lane == e, sstart, 0.0))
        sle = jnp.sum(jnp.where(lane == e, slots, 0.0))
        ble = jnp.sum(jnp.where(lane == e, blo, 0.0))
        ise = jnp.logical_and(lanef >= se, lanef < se + sle)
        eid = eid + jnp.where(ise, jnp.float32(e), 0.0)
        bid = bid + jnp.where(ise, ble + lanef - se, 0.0)
    laste = jnp.max(jnp.where(counts > 0.0, lanef, -1.0))
    lastb = jnp.max(jnp.where(counts > 0.0, bhi, -1.0))
    pad = lanef >= total
    eid = jnp.where(pad, laste, eid)
    bid = jnp.where(pad, lastb, bid)
    act = jnp.where(pad, 0.0, 1.0)
    tbl_ref[...] = jnp.concatenate(
        [eid, bid, act, offs_e], axis=0).astype(jnp.int32)

    full1 = offs_e + c_excl
    pos1 = jnp.sum(jnp.where(oh1 > 0.0, full1, 0.0), axis=-1, keepdims=True)
    pos2 = jnp.sum(jnp.where(oh2 > 0.0, full1 + oh1, 0.0), axis=-1,
                   keepdims=True)

    idx0_ref[...] = pos1.astype(jnp.int32)
    idx1_ref[...] = pos2.astype(jnp.int32)

    w1 = m1
    w2 = m2
    t2 = 2.0 * lax.broadcasted_iota(jnp.int32, (T, PCH), 0).astype(jnp.float32)
    for c in range(NPC):
        pvals = jnp.float32(c * PCH) + lax.broadcasted_iota(
            jnp.int32, (T, PCH), 1).astype(jnp.float32)
        eq0 = (pos1 == pvals).astype(jnp.float32)
        eq1 = (pos2 == pvals).astype(jnp.float32)
        perm_c = jnp.sum(eq0 * t2 + eq1 * (t2 + 1.0), axis=0)
        wts_c = jnp.sum(eq0 * w1 + eq1 * w2, axis=0)
        perm_ref[c, :] = perm_c.astype(jnp.int32)
        wts_ref[c, :] = wts_c


def _plan_call(router_logits):
    logits_pad = jnp.pad(router_logits, ((0, 0), (0, 128 - E)),
                         constant_values=-1e30)
    out_shapes = (
        jax.ShapeDtypeStruct((NPC, PCH), jnp.int32),
        jax.ShapeDtypeStruct((T, 1), jnp.int32),
        jax.ShapeDtypeStruct((T, 1), jnp.int32),
        jax.ShapeDtypeStruct((NPC, PCH), jnp.float32),
        jax.ShapeDtypeStruct((4, 128), jnp.int32),
    )
    return pl.pallas_call(
        _plan_body,
        out_shape=out_shapes,
        compiler_params=pltpu.CompilerParams(
            vmem_limit_bytes=120 * 1024 * 1024),
    )(logits_pad)


GCHP = 8


def _permute_body(x_hbm, perm_hbm, xs_hbm, idx_all, idx_ca, idx_cb,
                  bufa, bufb, sga, sgb, swa, swb):
    wid = lax.axis_index("s") * 2 + lax.axis_index("c")
    base = wid * RPW
    nch = RPW // GCHP
    pltpu.sync_copy(perm_hbm.at[pl.ds(base, RPW)], idx_all)

    idxs = (idx_ca, idx_cb)
    bufs = (bufa, bufb)
    gs = (sga, sgb)
    ws = (swa, swb)

    def _issue(j):
        s = j % 2
        idxs[s][...] = idx_all[pl.ds(j * GCHP, GCHP)]
        return pltpu.async_copy(x_hbm.at[idxs[s]], bufs[s], gs[s])

    pend = {0: _issue(0)}
    wr = [None, None]
    for j in range(nch):
        s = j % 2
        if j + 1 < nch:
            if wr[(j + 1) % 2] is not None:
                wr[(j + 1) % 2].wait()
            pend[j + 1] = _issue(j + 1)
        pend[j].wait()
        wr[s] = pltpu.async_copy(
            bufs[s], xs_hbm.at[pl.ds(base + j * GCHP, GCHP)], ws[s])
    for w in wr:
        if w is not None:
            w.wait()


def _permute_call(x, perm):
    return pl.kernel(
        _permute_body,
        out_type=jax.ShapeDtypeStruct((R, INTER), jnp.float32),
        mesh=plsc.VectorSubcoreMesh(core_axis_name="c", subcore_axis_name="s"),
        scratch_types=[
            pltpu.VMEM((RPW,), jnp.int32),
            pltpu.VMEM((GCHP,), jnp.int32),
            pltpu.VMEM((GCHP,), jnp.int32),
            pltpu.VMEM((GCHP, INTER), jnp.float32),
            pltpu.VMEM((GCHP, INTER), jnp.float32),
            pltpu.SemaphoreType.DMA,
            pltpu.SemaphoreType.DMA,
            pltpu.SemaphoreType.DMA,
            pltpu.SemaphoreType.DMA,
        ],
    )(x, perm)


def _gemm_body(tbl_ref, xs_ref, w_ref, wts_ref, out_ref):
    k = pl.program_id(0)
    g = pl.program_id(1)

    @pl.when(jnp.logical_and(k == 0, g == 0))
    def _init():
        out_ref[...] = jnp.zeros_like(out_ref)

    e = tbl_ref[0, g]
    b = tbl_ref[1, g]
    active = tbl_ref[2, g] > 0

    @pl.when(active)
    def _compute():
        lo = tbl_ref[3, e]
        hi = tbl_ref[3, e + 1]
        part = lax.dot_general(
            xs_ref[...], w_ref[0],
            (((1,), (0,)), ((), ())),
            preferred_element_type=jnp.float32,
        )
        rows = b * BLK + lax.broadcasted_iota(jnp.int32, (BLK, 1), 0)
        scale = jnp.where(
            jnp.logical_and(rows >= lo, rows < hi), wts_ref[...], 0.0)
        out_ref[pl.ds(b * BLK, BLK), :] += part * scale


def _gemm_call(tbl, xs, w, wts):
    grid_spec = pltpu.PrefetchScalarGridSpec(
        num_scalar_prefetch=1,
        grid=(NK, G),
        in_specs=[
            pl.BlockSpec((BLK, KBLK), lambda k, g, tbl: (tbl[1, g], k)),
            pl.BlockSpec((1, KBLK, HID), lambda k, g, tbl: (tbl[0, g], k, 0)),
            pl.BlockSpec((BLK, 1), lambda k, g, tbl: (tbl[1, g], 0)),
        ],
        out_specs=pl.BlockSpec((R, HID), lambda k, g, tbl: (0, 0)),
    )
    return pl.pallas_call(
        _gemm_body,
        grid_spec=grid_spec,
        out_shape=jax.ShapeDtypeStruct((R, HID), jnp.float32),
        compiler_params=pltpu.CompilerParams(
            dimension_semantics=("arbitrary", "arbitrary"),
            vmem_limit_bytes=100 * 1024 * 1024,
        ),
    )(tbl, xs, w, wts)


def _combine_body(y_hbm, idx0_hbm, idx1_hbm, out_hbm,
                  i0a, i1a, i0ca, i0cb, i1ca, i1cb,
                  b0a, b0b, b1a, b1b,
                  sg0a, sg0b, sg1a, sg1b, swa, swb):
    wid = lax.axis_index("s") * 2 + lax.axis_index("c")
    tb = wid * TPW
    nch = TPW // GCH
    pltpu.sync_copy(idx0_hbm.at[pl.ds(tb, TPW)], i0a)
    pltpu.sync_copy(idx1_hbm.at[pl.ds(tb, TPW)], i1a)

    idxs0 = (i0ca, i0cb)
    idxs1 = (i1ca, i1cb)
    bufs0 = (b0a, b0b)
    bufs1 = (b1a, b1b)
    gs0 = (sg0a, sg0b)
    gs1 = (sg1a, sg1b)
    ws = (swa, swb)

    def _issue(c):
        s = c % 2
        idxs0[s][...] = i0a[pl.ds(c * GCH, GCH)]
        idxs1[s][...] = i1a[pl.ds(c * GCH, GCH)]
        cp0 = pltpu.async_copy(y_hbm.at[idxs0[s]], bufs0[s], gs0[s])
        cp1 = pltpu.async_copy(y_hbm.at[idxs1[s]], bufs1[s], gs1[s])
        return cp0, cp1

    pend = {0: _issue(0)}
    wr = [None, None]
    for c in range(nch):
        s = c % 2
        if c + 1 < nch:
            if wr[(c + 1) % 2] is not None:
                wr[(c + 1) % 2].wait()
            pend[c + 1] = _issue(c + 1)
        cp0, cp1 = pend[c]
        cp0.wait()
        cp1.wait()

        @pl.loop(0, GCH)
        def _row(i):
            @pl.loop(0, HID // 16, unroll=8)
            def _vec(j):
                bufs0[s][i, pl.ds(j * 16, 16)] = (
                    bufs0[s][i, pl.ds(j * 16, 16)]
                    + bufs1[s][i, pl.ds(j * 16, 16)])

        wr[s] = pltpu.async_copy(
            bufs0[s], out_hbm.at[pl.ds(tb + c * GCH, GCH)], ws[s])
    for w in wr:
        if w is not None:
            w.wait()


def _combine_call(y, idx0, idx1):
    return pl.kernel(
        _combine_body,
        out_type=jax.ShapeDtypeStruct((T, HID), jnp.float32),
        mesh=plsc.VectorSubcoreMesh(core_axis_name="c", subcore_axis_name="s"),
        scratch_types=[
            pltpu.VMEM((TPW,), jnp.int32),
            pltpu.VMEM((TPW,), jnp.int32),
            pltpu.VMEM((GCH,), jnp.int32),
            pltpu.VMEM((GCH,), jnp.int32),
            pltpu.VMEM((GCH,), jnp.int32),
            pltpu.VMEM((GCH,), jnp.int32),
            pltpu.VMEM((GCH, HID), jnp.float32),
            pltpu.VMEM((GCH, HID), jnp.float32),
            pltpu.VMEM((GCH, HID), jnp.float32),
            pltpu.VMEM((GCH, HID), jnp.float32),
            pltpu.SemaphoreType.DMA,
            pltpu.SemaphoreType.DMA,
            pltpu.SemaphoreType.DMA,
            pltpu.SemaphoreType.DMA,
            pltpu.SemaphoreType.DMA,
            pltpu.SemaphoreType.DMA,
        ],
    )(y, idx0, idx1)


def kernel(intermediate_states, w, router_logits):
    perm2, idx0, idx1, wts2, tbl = _plan_call(router_logits)
    perm = perm2.reshape(R)
    wts = wts2.reshape(R, 1)
    xs = _permute_call(intermediate_states, perm)
    y = _gemm_call(tbl, xs, w, wts)
    return _combine_call(y, idx0.reshape(T), idx1.reshape(T))


# speedup vs baseline: 1.0630x; 1.0630x over previous
import functools

import jax
import jax.numpy as jnp
from jax import lax
from jax.experimental import pallas as pl
from jax.experimental.pallas import tpu as pltpu
from jax.experimental.pallas import tpu_sc as plsc

T = 2048
K_TOP = 2
E = 8
INTER = 4096
HID = 1024
R = T * K_TOP

BLK = 256
NB = R // BLK
G = NB + E
KBLK = 4096
NK = INTER // KBLK
PCH = 512
NPC = R // PCH

NW = 32
RPW = R // NW
TPW = T // NW
GCH = 16


def _plan_body(logits_ref, perm_ref, idx0_ref, idx1_ref, wts_ref, tbl_ref):
    logits = logits_ref[...]
    col = lax.broadcasted_iota(jnp.int32, (T, 128), 1)
    m = jnp.max(logits, axis=-1, keepdims=True)
    ex = jnp.exp(logits - m)
    probs = ex / jnp.sum(ex, axis=-1, keepdims=True)

    m1 = jnp.max(probs, axis=-1, keepdims=True)
    i1 = jnp.min(jnp.where(probs == m1, col, 128), axis=-1, keepdims=True)
    probs2 = jnp.where(col == i1, -1.0, probs)
    m2 = jnp.max(probs2, axis=-1, keepdims=True)
    i2 = jnp.min(jnp.where(probs2 == m2, col, 128), axis=-1, keepdims=True)

    oh1 = (col == i1).astype(jnp.float32)
    oh2 = (col == i2).astype(jnp.float32)
    oh = oh1 + oh2

    nt = T // 128
    ohr = oh.reshape(nt, 128, 128)
    r1 = lax.broadcasted_iota(jnp.int32, (128, 128), 0)
    c1 = lax.broadcasted_iota(jnp.int32, (128, 128), 1)
    ltri1 = jnp.broadcast_to((r1 >= c1).astype(jnp.float32), (nt, 128, 128))
    ci = lax.dot_general(ltri1, ohr, (((2,), (1,)), ((0,), (0,))),
                         preferred_element_type=jnp.float32)
    tile_sum = jnp.sum(ohr, axis=1)
    r2 = lax.broadcasted_iota(jnp.int32, (nt, nt), 0)
    c2 = lax.broadcasted_iota(jnp.int32, (nt, nt), 1)
    sltri2 = (r2 > c2).astype(jnp.float32)
    tile_pre = lax.dot(sltri2, tile_sum,
                       preferred_element_type=jnp.float32)
    c_incl = (ci + tile_pre[:, None, :]).reshape(T, 128)
    c_excl = c_incl - oh

    counts = jnp.sum(oh, axis=0, keepdims=True)

    lr = lax.broadcasted_iota(jnp.int32, (128, 128), 0)
    lc = lax.broadcasted_iota(jnp.int32, (128, 128), 1)
    sl = (lr < lc).astype(jnp.float32)
    offs_e = lax.dot(counts, sl, precision=lax.Precision.HIGHEST,
                     preferred_element_type=jnp.float32)

    lane = lax.broadcasted_iota(jnp.int32, (1, 128), 1)
    lanef = lane.astype(jnp.float32)
    blk_f = jnp.float32(BLK)
    lo_f = offs_e
    hi_f = offs_e + counts
    blo = jnp.floor(lo_f / blk_f)
    bhi = jnp.floor((hi_f - 1.0) / blk_f)
    slots = jnp.where(counts > 0.0, bhi - blo + 1.0, 0.0)
    sstart = lax.dot(slots, sl, precision=lax.Precision.HIGHEST,
                     preferred_element_type=jnp.float32)
    total = jnp.sum(slots)
    eid = jnp.zeros((1, 128), jnp.float32)
    bid = jnp.zeros((1, 128), jnp.float32)
    for e in range(E):
        se = jnp.sum(jnp.where(lane == e, sstart, 0.0))
        sle = jnp.sum(jnp.where(lane == e, slots, 0.0))
        ble = jnp.sum(jnp.where(lane == e, blo, 0.0))
        ise = jnp.logical_and(lanef >= se, lanef < se + sle)
        eid = eid + jnp.where(ise, jnp.float32(e), 0.0)
        bid = bid + jnp.where(ise, ble + lanef - se, 0.0)
    laste = jnp.max(jnp.where(counts > 0.0, lanef, -1.0))
    lastb = jnp.max(jnp.where(counts > 0.0, bhi, -1.0))
    pad = lanef >= total
    eid = jnp.where(pad, laste, eid)
    bid = jnp.where(pad, lastb, bid)
    act = jnp.where(pad, 0.0, 1.0)
    tbl_ref[...] = jnp.concatenate(
        [eid, bid, act, offs_e], axis=0).astype(jnp.int32)

    full1 = offs_e + c_excl
    pos1 = jnp.sum(jnp.where(oh1 > 0.0, full1, 0.0), axis=-1, keepdims=True)
    pos2 = jnp.sum(jnp.where(oh2 > 0.0, full1 + oh1, 0.0), axis=-1,
                   keepdims=True)

    idx0_ref[...] = pos1.astype(jnp.int32)
    idx1_ref[...] = pos2.astype(jnp.int32)

    w1 = m1
    w2 = m2
    t2 = 2.0 * lax.broadcasted_iota(jnp.int32, (T, PCH), 0).astype(jnp.float32)
    for c in range(NPC):
        pvals = jnp.float32(c * PCH) + lax.broadcasted_iota(
            jnp.int32, (T, PCH), 1).astype(jnp.float32)
        eq0 = (pos1 == pvals).astype(jnp.float32)
        eq1 = (pos2 == pvals).astype(jnp.float32)
        perm_c = jnp.sum(eq0 * t2 + eq1 * (t2 + 1.0), axis=0)
        wts_c = jnp.sum(eq0 * w1 + eq1 * w2, axis=0)
        perm_ref[c, :] = perm_c.astype(jnp.int32)
        wts_ref[c, :] = wts_c


def _plan_call(router_logits):
    logits_pad = jnp.pad(router_logits, ((0, 0), (0, 128 - E)),
                         constant_values=-1e30)
    out_shapes = (
        jax.ShapeDtypeStruct((NPC, PCH), jnp.int32),
        jax.ShapeDtypeStruct((T, 1), jnp.int32),
        jax.ShapeDtypeStruct((T, 1), jnp.int32),
        jax.ShapeDtypeStruct((NPC, PCH), jnp.float32),
        jax.ShapeDtypeStruct((4, 128), jnp.int32),
    )
    return pl.pallas_call(
        _plan_body,
        out_shape=out_shapes,
        compiler_params=pltpu.CompilerParams(
            vmem_limit_bytes=120 * 1024 * 1024),
    )(logits_pad)


GCHP = 8


def _permute_body(x_hbm, perm_hbm, xs_hbm, idx_all, idx_ca, idx_cb,
                  bufa, bufb, sga, sgb, swa, swb):
    wid = lax.axis_index("s") * 2 + lax.axis_index("c")
    base = wid * RPW
    nch = RPW // GCHP
    pltpu.sync_copy(perm_hbm.at[pl.ds(base, RPW)], idx_all)

    idxs = (idx_ca, idx_cb)
    bufs = (bufa, bufb)
    gs = (sga, sgb)
    ws = (swa, swb)

    def _issue(j):
        s = j % 2
        idxs[s][...] = idx_all[pl.ds(j * GCHP, GCHP)]
        return pltpu.async_copy(x_hbm.at[idxs[s]], bufs[s], gs[s])

    pend = {0: _issue(0)}
    wr = [None, None]
    for j in range(nch):
        s = j % 2
        if j + 1 < nch:
            if wr[(j + 1) % 2] is not None:
                wr[(j + 1) % 2].wait()
            pend[j + 1] = _issue(j + 1)
        pend[j].wait()
        wr[s] = pltpu.async_copy(
            bufs[s], xs_hbm.at[pl.ds(base + j * GCHP, GCHP)], ws[s])
    for w in wr:
        if w is not None:
            w.wait()


def _permute_call(x, perm):
    return pl.kernel(
        _permute_body,
        out_type=jax.ShapeDtypeStruct((R, INTER), jnp.float32),
        mesh=plsc.VectorSubcoreMesh(core_axis_name="c", subcore_axis_name="s"),
        scratch_types=[
            pltpu.VMEM((RPW,), jnp.int32),
            pltpu.VMEM((GCHP,), jnp.int32),
            pltpu.VMEM((GCHP,), jnp.int32),
            pltpu.VMEM((GCHP, INTER), jnp.float32),
            pltpu.VMEM((GCHP, INTER), jnp.float32),
            pltpu.SemaphoreType.DMA,
            pltpu.SemaphoreType.DMA,
            pltpu.SemaphoreType.DMA,
            pltpu.SemaphoreType.DMA,
        ],
    )(x, perm)


def _gemm_body(tbl_ref, xs_ref, w_ref, wts_ref, out_ref):
    k = pl.program_id(0)
    g = pl.program_id(1)

    @pl.when(jnp.logical_and(k == 0, g == 0))
    def _init():
        out_ref[...] = jnp.zeros_like(out_ref)

    e = tbl_ref[0, g]
    b = tbl_ref[1, g]
    active = tbl_ref[2, g] > 0

    @pl.when(active)
    def _compute():
        lo = tbl_ref[3, e]
        hi = tbl_ref[3, e + 1]
        part = lax.dot_general(
            xs_ref[...].astype(jnp.bfloat16), w_ref[0].astype(jnp.bfloat16),
            (((1,), (0,)), ((), ())),
            preferred_element_type=jnp.float32,
        )
        rows = b * BLK + lax.broadcasted_iota(jnp.int32, (BLK, 1), 0)
        scale = jnp.where(
            jnp.logical_and(rows >= lo, rows < hi), wts_ref[...], 0.0)
        out_ref[pl.ds(b * BLK, BLK), :] += part * scale


def _gemm_call(tbl, xs, w, wts):
    grid_spec = pltpu.PrefetchScalarGridSpec(
        num_scalar_prefetch=1,
        grid=(NK, G),
        in_specs=[
            pl.BlockSpec((BLK, KBLK), lambda k, g, tbl: (tbl[1, g], k)),
            pl.BlockSpec((1, KBLK, HID), lambda k, g, tbl: (tbl[0, g], k, 0)),
            pl.BlockSpec((BLK, 1), lambda k, g, tbl: (tbl[1, g], 0)),
        ],
        out_specs=pl.BlockSpec((R, HID), lambda k, g, tbl: (0, 0)),
    )
    return pl.pallas_call(
        _gemm_body,
        grid_spec=grid_spec,
        out_shape=jax.ShapeDtypeStruct((R, HID), jnp.float32),
        compiler_params=pltpu.CompilerParams(
            dimension_semantics=("arbitrary", "arbitrary"),
            vmem_limit_bytes=100 * 1024 * 1024,
        ),
    )(tbl, xs, w, wts)


def _combine_body(y_hbm, idx0_hbm, idx1_hbm, out_hbm,
                  i0a, i1a, i0ca, i0cb, i1ca, i1cb,
                  b0a, b0b, b1a, b1b,
                  sg0a, sg0b, sg1a, sg1b, swa, swb):
    wid = lax.axis_index("s") * 2 + lax.axis_index("c")
    tb = wid * TPW
    nch = TPW // GCH
    pltpu.sync_copy(idx0_hbm.at[pl.ds(tb, TPW)], i0a)
    pltpu.sync_copy(idx1_hbm.at[pl.ds(tb, TPW)], i1a)

    idxs0 = (i0ca, i0cb)
    idxs1 = (i1ca, i1cb)
    bufs0 = (b0a, b0b)
    bufs1 = (b1a, b1b)
    gs0 = (sg0a, sg0b)
    gs1 = (sg1a, sg1b)
    ws = (swa, swb)

    def _issue(c):
        s = c % 2
        idxs0[s][...] = i0a[pl.ds(c * GCH, GCH)]
        idxs1[s][...] = i1a[pl.ds(c * GCH, GCH)]
        cp0 = pltpu.async_copy(y_hbm.at[idxs0[s]], bufs0[s], gs0[s])
        cp1 = pltpu.async_copy(y_hbm.at[idxs1[s]], bufs1[s], gs1[s])
        return cp0, cp1

    pend = {0: _issue(0)}
    wr = [None, None]
    for c in range(nch):
        s = c % 2
        if c + 1 < nch:
            if wr[(c + 1) % 2] is not None:
                wr[(c + 1) % 2].wait()
            pend[c + 1] = _issue(c + 1)
        cp0, cp1 = pend[c]
        cp0.wait()
        cp1.wait()

        @pl.loop(0, GCH)
        def _row(i):
            @pl.loop(0, HID // 16, unroll=8)
            def _vec(j):
                bufs0[s][i, pl.ds(j * 16, 16)] = (
                    bufs0[s][i, pl.ds(j * 16, 16)]
                    + bufs1[s][i, pl.ds(j * 16, 16)])

        wr[s] = pltpu.async_copy(
            bufs0[s], out_hbm.at[pl.ds(tb + c * GCH, GCH)], ws[s])
    for w in wr:
        if w is not None:
            w.wait()


def _combine_call(y, idx0, idx1):
    return pl.kernel(
        _combine_body,
        out_type=jax.ShapeDtypeStruct((T, HID), jnp.float32),
        mesh=plsc.VectorSubcoreMesh(core_axis_name="c", subcore_axis_name="s"),
        scratch_types=[
            pltpu.VMEM((TPW,), jnp.int32),
            pltpu.VMEM((TPW,), jnp.int32),
            pltpu.VMEM((GCH,), jnp.int32),
            pltpu.VMEM((GCH,), jnp.int32),
            pltpu.VMEM((GCH,), jnp.int32),
            pltpu.VMEM((GCH,), jnp.int32),
            pltpu.VMEM((GCH, HID), jnp.float32),
            pltpu.VMEM((GCH, HID), jnp.float32),
            pltpu.VMEM((GCH, HID), jnp.float32),
            pltpu.VMEM((GCH, HID), jnp.float32),
            pltpu.SemaphoreType.DMA,
            pltpu.SemaphoreType.DMA,
            pltpu.SemaphoreType.DMA,
            pltpu.SemaphoreType.DMA,
            pltpu.SemaphoreType.DMA,
            pltpu.SemaphoreType.DMA,
        ],
    )(y, idx0, idx1)


def kernel(intermediate_states, w, router_logits):
    perm2, idx0, idx1, wts2, tbl = _plan_call(router_logits)
    perm = perm2.reshape(R)
    wts = wts2.reshape(R, 1)
    xs = _permute_call(intermediate_states, perm)
    y = _gemm_call(tbl, xs, w, wts)
    return _combine_call(y, idx0.reshape(T), idx1.reshape(T))


# speedup vs baseline: 1.0635x; 1.0005x over previous
import functools

import jax
import jax.numpy as jnp
from jax import lax
from jax.experimental import pallas as pl
from jax.experimental.pallas import tpu as pltpu
from jax.experimental.pallas import tpu_sc as plsc

T = 2048
K_TOP = 2
E = 8
INTER = 4096
HID = 1024
R = T * K_TOP

BLK = 256
NB = R // BLK
G = NB + E
KBLK = 4096
NK = INTER // KBLK
PCH = 512
NPC = R // PCH

NW = 32
RPW = R // NW
TPW = T // NW
GCH = 16


def _plan_body(logits_ref, perm_ref, idx0_ref, idx1_ref, wts_ref, tbl_ref):
    logits = logits_ref[...]
    col = lax.broadcasted_iota(jnp.int32, (T, 128), 1)
    m = jnp.max(logits, axis=-1, keepdims=True)
    ex = jnp.exp(logits - m)
    probs = ex / jnp.sum(ex, axis=-1, keepdims=True)

    m1 = jnp.max(probs, axis=-1, keepdims=True)
    i1 = jnp.min(jnp.where(probs == m1, col, 128), axis=-1, keepdims=True)
    probs2 = jnp.where(col == i1, -1.0, probs)
    m2 = jnp.max(probs2, axis=-1, keepdims=True)
    i2 = jnp.min(jnp.where(probs2 == m2, col, 128), axis=-1, keepdims=True)

    oh1 = (col == i1).astype(jnp.float32)
    oh2 = (col == i2).astype(jnp.float32)
    oh = oh1 + oh2

    nt = T // 128
    ohr = oh.reshape(nt, 128, 128)
    r1 = lax.broadcasted_iota(jnp.int32, (128, 128), 0)
    c1 = lax.broadcasted_iota(jnp.int32, (128, 128), 1)
    ltri1 = jnp.broadcast_to((r1 >= c1).astype(jnp.float32), (nt, 128, 128))
    ci = lax.dot_general(ltri1, ohr, (((2,), (1,)), ((0,), (0,))),
                         preferred_element_type=jnp.float32)
    tile_sum = jnp.sum(ohr, axis=1)
    r2 = lax.broadcasted_iota(jnp.int32, (nt, nt), 0)
    c2 = lax.broadcasted_iota(jnp.int32, (nt, nt), 1)
    sltri2 = (r2 > c2).astype(jnp.float32)
    tile_pre = lax.dot(sltri2, tile_sum,
                       preferred_element_type=jnp.float32)
    c_incl = (ci + tile_pre[:, None, :]).reshape(T, 128)
    c_excl = c_incl - oh

    counts = jnp.sum(oh, axis=0, keepdims=True)

    lr = lax.broadcasted_iota(jnp.int32, (128, 128), 0)
    lc = lax.broadcasted_iota(jnp.int32, (128, 128), 1)
    sl = (lr < lc).astype(jnp.float32)
    offs_e = lax.dot(counts, sl, precision=lax.Precision.HIGHEST,
                     preferred_element_type=jnp.float32)

    lane = lax.broadcasted_iota(jnp.int32, (1, 128), 1)
    lanef = lane.astype(jnp.float32)
    blk_f = jnp.float32(BLK)
    lo_f = offs_e
    hi_f = offs_e + counts
    blo = jnp.floor(lo_f / blk_f)
    bhi = jnp.floor((hi_f - 1.0) / blk_f)
    slots = jnp.where(counts > 0.0, bhi - blo + 1.0, 0.0)
    sstart = lax.dot(slots, sl, precision=lax.Precision.HIGHEST,
                     preferred_element_type=jnp.float32)
    total = jnp.sum(slots)
    eid = jnp.zeros((1, 128), jnp.float32)
    bid = jnp.zeros((1, 128), jnp.float32)
    for e in range(E):
        se = jnp.sum(jnp.where(lane == e, sstart, 0.0))
        sle = jnp.sum(jnp.where(lane == e, slots, 0.0))
        ble = jnp.sum(jnp.where(lane == e, blo, 0.0))
        ise = jnp.logical_and(lanef >= se, lanef < se + sle)
        eid = eid + jnp.where(ise, jnp.float32(e), 0.0)
        bid = bid + jnp.where(ise, ble + lanef - se, 0.0)
    laste = jnp.max(jnp.where(counts > 0.0, lanef, -1.0))
    lastb = jnp.max(jnp.where(counts > 0.0, bhi, -1.0))
    pad = lanef >= total
    eid = jnp.where(pad, laste, eid)
    bid = jnp.where(pad, lastb, bid)
    act = jnp.where(pad, 0.0, 1.0)
    tbl_ref[...] = jnp.concatenate(
        [eid, bid, act, offs_e], axis=0).astype(jnp.int32)

    full1 = offs_e + c_excl
    pos1 = jnp.sum(jnp.where(oh1 > 0.0, full1, 0.0), axis=-1, keepdims=True)
    pos2 = jnp.sum(jnp.where(oh2 > 0.0, full1 + oh1, 0.0), axis=-1,
                   keepdims=True)

    idx0_ref[...] = pos1.astype(jnp.int32)
    idx1_ref[...] = pos2.astype(jnp.int32)

    w1 = m1
    w2 = m2
    t2 = 2.0 * lax.broadcasted_iota(jnp.int32, (T, PCH), 0).astype(jnp.float32)
    for c in range(NPC):
        pvals = jnp.float32(c * PCH) + lax.broadcasted_iota(
            jnp.int32, (T, PCH), 1).astype(jnp.float32)
        eq0 = (pos1 == pvals).astype(jnp.float32)
        eq1 = (pos2 == pvals).astype(jnp.float32)
        perm_c = jnp.sum(eq0 * t2 + eq1 * (t2 + 1.0), axis=0)
        wts_c = jnp.sum(eq0 * w1 + eq1 * w2, axis=0)
        perm_ref[c, :] = perm_c.astype(jnp.int32)
        wts_ref[c, :] = wts_c


def _plan_call(router_logits):
    logits_pad = jnp.pad(router_logits, ((0, 0), (0, 128 - E)),
                         constant_values=-1e30)
    out_shapes = (
        jax.ShapeDtypeStruct((NPC, PCH), jnp.int32),
        jax.ShapeDtypeStruct((T, 1), jnp.int32),
        jax.ShapeDtypeStruct((T, 1), jnp.int32),
        jax.ShapeDtypeStruct((NPC, PCH), jnp.float32),
        jax.ShapeDtypeStruct((4, 128), jnp.int32),
    )
    return pl.pallas_call(
        _plan_body,
        out_shape=out_shapes,
        compiler_params=pltpu.CompilerParams(
            vmem_limit_bytes=120 * 1024 * 1024),
    )(logits_pad)


GCHP = 8


def _permute_body(x_hbm, perm_hbm, xs_hbm, idx_all, idx_ca, idx_cb,
                  bufa, bufb, sga, sgb, swa, swb):
    wid = lax.axis_index("s") * 2 + lax.axis_index("c")
    base = wid * RPW
    nch = RPW // GCHP
    pltpu.sync_copy(perm_hbm.at[pl.ds(base, RPW)], idx_all)

    idxs = (idx_ca, idx_cb)
    bufs = (bufa, bufb)
    gs = (sga, sgb)
    ws = (swa, swb)

    def _issue(j):
        s = j % 2
        idxs[s][...] = idx_all[pl.ds(j * GCHP, GCHP)]
        return pltpu.async_copy(x_hbm.at[idxs[s]], bufs[s], gs[s])

    pend = {0: _issue(0)}
    wr = [None, None]
    for j in range(nch):
        s = j % 2
        if j + 1 < nch:
            if wr[(j + 1) % 2] is not None:
                wr[(j + 1) % 2].wait()
            pend[j + 1] = _issue(j + 1)
        pend[j].wait()
        wr[s] = pltpu.async_copy(
            bufs[s], xs_hbm.at[pl.ds(base + j * GCHP, GCHP)], ws[s])
    for w in wr:
        if w is not None:
            w.wait()


def _permute_call(x, perm):
    return pl.kernel(
        _permute_body,
        out_type=jax.ShapeDtypeStruct((R, INTER), jnp.float32),
        mesh=plsc.VectorSubcoreMesh(core_axis_name="c", subcore_axis_name="s"),
        scratch_types=[
            pltpu.VMEM((RPW,), jnp.int32),
            pltpu.VMEM((GCHP,), jnp.int32),
            pltpu.VMEM((GCHP,), jnp.int32),
            pltpu.VMEM((GCHP, INTER), jnp.float32),
            pltpu.VMEM((GCHP, INTER), jnp.float32),
            pltpu.SemaphoreType.DMA,
            pltpu.SemaphoreType.DMA,
            pltpu.SemaphoreType.DMA,
            pltpu.SemaphoreType.DMA,
        ],
    )(x, perm)


def _gemm_body(tbl_ref, xs_ref, w_ref, wts_ref, out_ref):
    k = pl.program_id(0)
    g = pl.program_id(1)

    @pl.when(jnp.logical_and(k == 0, g == 0))
    def _init():
        out_ref[...] = jnp.zeros_like(out_ref)

    e = tbl_ref[0, g]
    b = tbl_ref[1, g]
    active = tbl_ref[2, g] > 0

    @pl.when(active)
    def _compute():
        lo = tbl_ref[3, e]
        hi = tbl_ref[3, e + 1]
        part = lax.dot_general(
            xs_ref[...], w_ref[0],
            (((1,), (0,)), ((), ())),
            preferred_element_type=jnp.float32,
        )
        rows = b * BLK + lax.broadcasted_iota(jnp.int32, (BLK, 1), 0)
        scale = jnp.where(
            jnp.logical_and(rows >= lo, rows < hi), wts_ref[...], 0.0)
        out_ref[pl.ds(b * BLK, BLK), :] += part * scale


def _gemm_call(tbl, xs, w, wts):
    grid_spec = pltpu.PrefetchScalarGridSpec(
        num_scalar_prefetch=1,
        grid=(NK, G),
        in_specs=[
            pl.BlockSpec((BLK, KBLK), lambda k, g, tbl: (tbl[1, g], k)),
            pl.BlockSpec((1, KBLK, HID), lambda k, g, tbl: (tbl[0, g], k, 0)),
            pl.BlockSpec((BLK, 1), lambda k, g, tbl: (tbl[1, g], 0)),
        ],
        out_specs=pl.BlockSpec((R, HID), lambda k, g, tbl: (0, 0)),
    )
    return pl.pallas_call(
        _gemm_body,
        grid_spec=grid_spec,
        out_shape=jax.ShapeDtypeStruct((R, HID), jnp.float32),
        compiler_params=pltpu.CompilerParams(
            dimension_semantics=("arbitrary", "arbitrary"),
            vmem_limit_bytes=100 * 1024 * 1024,
        ),
    )(tbl, xs, w, wts)


def _combine_body(y_hbm, idx0_hbm, idx1_hbm, out_hbm,
                  i0a, i1a, i0ca, i0cb, i1ca, i1cb,
                  b0a, b0b, b1a, b1b,
                  sg0a, sg0b, sg1a, sg1b, swa, swb):
    wid = lax.axis_index("s") * 2 + lax.axis_index("c")
    tb = wid * TPW
    nch = TPW // GCH
    pltpu.sync_copy(idx0_hbm.at[pl.ds(tb, TPW)], i0a)
    pltpu.sync_copy(idx1_hbm.at[pl.ds(tb, TPW)], i1a)

    idxs0 = (i0ca, i0cb)
    idxs1 = (i1ca, i1cb)
    bufs0 = (b0a, b0b)
    bufs1 = (b1a, b1b)
    gs0 = (sg0a, sg0b)
    gs1 = (sg1a, sg1b)
    ws = (swa, swb)

    def _issue(c):
        s = c % 2
        idxs0[s][...] = i0a[pl.ds(c * GCH, GCH)]
        idxs1[s][...] = i1a[pl.ds(c * GCH, GCH)]
        cp0 = pltpu.async_copy(y_hbm.at[idxs0[s]], bufs0[s], gs0[s])
        cp1 = pltpu.async_copy(y_hbm.at[idxs1[s]], bufs1[s], gs1[s])
        return cp0, cp1

    pend = {0: _issue(0)}
    wr = [None, None]
    for c in range(nch):
        s = c % 2
        if c + 1 < nch:
            if wr[(c + 1) % 2] is not None:
                wr[(c + 1) % 2].wait()
            pend[c + 1] = _issue(c + 1)
        cp0, cp1 = pend[c]
        cp0.wait()
        cp1.wait()

        @pl.loop(0, GCH)
        def _row(i):
            @pl.loop(0, HID // 16, unroll=8)
            def _vec(j):
                bufs0[s][i, pl.ds(j * 16, 16)] = (
                    bufs0[s][i, pl.ds(j * 16, 16)]
                    + bufs1[s][i, pl.ds(j * 16, 16)])

        wr[s] = pltpu.async_copy(
            bufs0[s], out_hbm.at[pl.ds(tb + c * GCH, GCH)], ws[s])
    for w in wr:
        if w is not None:
            w.wait()


def _combine_call(y, idx0, idx1):
    return pl.kernel(
        _combine_body,
        out_type=jax.ShapeDtypeStruct((T, HID), jnp.float32),
        mesh=plsc.VectorSubcoreMesh(core_axis_name="c", subcore_axis_name="s"),
        scratch_types=[
            pltpu.VMEM((TPW,), jnp.int32),
            pltpu.VMEM((TPW,), jnp.int32),
            pltpu.VMEM((GCH,), jnp.int32),
            pltpu.VMEM((GCH,), jnp.int32),
            pltpu.VMEM((GCH,), jnp.int32),
            pltpu.VMEM((GCH,), jnp.int32),
            pltpu.VMEM((GCH, HID), jnp.float32),
            pltpu.VMEM((GCH, HID), jnp.float32),
            pltpu.VMEM((GCH, HID), jnp.float32),
            pltpu.VMEM((GCH, HID), jnp.float32),
            pltpu.SemaphoreType.DMA,
            pltpu.SemaphoreType.DMA,
            pltpu.SemaphoreType.DMA,
            pltpu.SemaphoreType.DMA,
            pltpu.SemaphoreType.DMA,
            pltpu.SemaphoreType.DMA,
        ],
    )(y, idx0, idx1)


def kernel(intermediate_states, w, router_logits):
    perm2, idx0, idx1, wts2, tbl = _plan_call(router_logits)
    perm = perm2.reshape(R)
    wts = wts2.reshape(R, 1)
    xs = _permute_call(intermediate_states, perm)
    y = _gemm_call(tbl, xs, w, wts)
    return _combine_call(y, idx0.reshape(T), idx1.reshape(T))
